# Initial kernel scaffold; baseline (speedup 1.0000x reference)
import jax
import jax.numpy as jnp
from jax.experimental import pallas as pl


def kernel(x, edge_index, W_rel1, b_rel1, W_root1, g1, be1, W_rel2, b_rel2, W_root2, g2, be2, W_rel3, b_rel3, W_root3, g3, be3, Wh1, bh1, Wh2, bh2):
    raise NotImplementedError("write your pallas kernel here")


# speedup vs baseline: 4.1265x; 4.1265x over previous
import functools

import jax
import jax.numpy as jnp
from jax import lax
from jax.experimental import pallas as pl
from jax.experimental.pallas import tpu as pltpu
from jax.experimental.pallas import tpu_sc as plsc

N = 50000
E = 800000
NSUB = 16
G = 128
EB = 2000


def _ceil(a, b):
    return -(-a // b)


@functools.partial(jax.jit, static_argnames=("d", "C"))
def _segsum(table, src, dst, d=128, C=8):
    NB = 128 * _ceil(N, 128 * C)
    NPAD = NB * C
    CPS = C // 2
    EW = E // NSUB
    NBLK = EW // EB
    VPB = EB // 16
    ACC_ROWS = NB + 128
    ZRP = ACC_ROWS // NSUB
    RP = NB // NSUB

    mesh = plsc.VectorSubcoreMesh(core_axis_name="c", subcore_axis_name="s")

    @functools.partial(
        pl.kernel,
        out_type=jax.ShapeDtypeStruct((NPAD, d), jnp.float32),
        mesh=mesh,
        compiler_params=pltpu.CompilerParams(needs_layout_passes=False),
        scratch_types=[
            pltpu.VMEM_SHARED((ACC_ROWS, d), jnp.float32),
            pltpu.VMEM((EB,), jnp.int32),
            pltpu.VMEM((EB,), jnp.int32),
            pltpu.VMEM((EB + G,), jnp.int32),
            pltpu.VMEM((EB + G,), jnp.int32),
            pltpu.VMEM((1, G), jnp.int32),
            pltpu.VMEM((G, d), jnp.float32),
            pltpu.VMEM((G, d), jnp.float32),
            pltpu.SemaphoreType.DMA,
        ],
    )
    def seg(table_h, src_h, dst_h, out_h, acc, dst_buf, src_buf, src_list,
            loc_list, grp_idx, rows, zeros, sem):
        cid = lax.axis_index("c")
        sid = lax.axis_index("s")
        iota = lax.iota(jnp.int32, 16)
        z16 = jnp.zeros((16,), jnp.float32)

        def zero_row(r, _):
            for j in range(d // 16):
                zeros[r, pl.ds(j * 16, 16)] = z16
            return 0

        lax.fori_loop(0, G, zero_row, 0)

        pad_src = cid * 256 + sid * 16 + iota
        pad_loc = NB + iota

        for k in range(CPS):
            lo = (cid * CPS + k) * NB
            hi = lo + NB

            zbase = sid * ZRP
            for t in range(ZRP // G):
                pltpu.sync_copy(zeros, acc.at[pl.ds(zbase + t * G, G)])
            if ZRP % G:
                pltpu.sync_copy(zeros.at[pl.ds(0, ZRP % G)],
                                acc.at[pl.ds(zbase + (ZRP // G) * G, ZRP % G)])
            plsc.subcore_barrier()

            def blk_body(blk, _):
                base = sid * EW + blk * EB
                pltpu.sync_copy(dst_h.at[pl.ds(base, EB)], dst_buf)
                pltpu.sync_copy(src_h.at[pl.ds(base, EB)], src_buf)

                def scan_body(i, cnt):
                    dv = dst_buf[pl.ds(i * 16, 16)]
                    sv = src_buf[pl.ds(i * 16, 16)]
                    m = (dv >= lo) & (dv < hi)
                    mi = m.astype(jnp.int32)
                    pos = cnt + plsc.cumsum(mi) - 1
                    plsc.store_scatter(src_list, [pos], sv, mask=m)
                    plsc.store_scatter(loc_list, [pos], dv - lo, mask=m)
                    return cnt + jnp.sum(mi)

                kcnt = lax.fori_loop(0, VPB, scan_body, 0)

                for t in range(G // 16):
                    src_list[pl.ds(kcnt + t * 16, 16)] = pad_src
                    loc_list[pl.ds(kcnt + t * 16, 16)] = pad_loc

                def grp_body(g, _):
                    gb = g * G
                    for t in range(G // 16):
                        grp_idx[0, pl.ds(t * 16, 16)] = (
                            loc_list[pl.ds(gb + t * 16, 16)])
                    pltpu.async_copy(
                        table_h.at[src_list.at[pl.ds(gb, G)]], rows,
                        sem).wait()
                    pltpu.sync_copy(rows, acc.at[grp_idx.at[0]], add=True)
                    return 0

                lax.fori_loop(0, (kcnt + G - 1) // G, grp_body, 0)
                return 0

            lax.fori_loop(0, NBLK, blk_body, 0)
            plsc.subcore_barrier()

            pltpu.sync_copy(acc.at[pl.ds(sid * RP, RP)],
                            out_h.at[pl.ds(lo + sid * RP, RP)])
            plsc.subcore_barrier()

    return seg(table, src, dst)


_R = 2000


@jax.jit
def _mm_stats(agg, xin, wr, wo, b):
    dout = wr.shape[1]

    def body(agg_ref, x_ref, wr_ref, wo_ref, b_ref, y_ref, st_ref):
        yb = jnp.dot(agg_ref[...], wr_ref[...],
                     preferred_element_type=jnp.float32)
        yb = yb + jnp.dot(x_ref[...], wo_ref[...],
                          preferred_element_type=jnp.float32)
        yb = yb + b_ref[...]
        y_ref[...] = yb
        s = jnp.sum(yb, axis=0, keepdims=True)
        s2 = jnp.sum(yb * yb, axis=0, keepdims=True)
        st = jnp.concatenate(
            [s, s2, jnp.zeros((6, dout), jnp.float32)], axis=0)

        @pl.when(pl.program_id(0) == 0)
        def _():
            st_ref[...] = st

        @pl.when(pl.program_id(0) > 0)
        def _():
            st_ref[...] = st_ref[...] + st

    din = agg.shape[1]
    return pl.pallas_call(
        body,
        grid=(N // _R,),
        in_specs=[
            pl.BlockSpec((_R, din), lambda i: (i, 0)),
            pl.BlockSpec((_R, din), lambda i: (i, 0)),
            pl.BlockSpec((din, dout), lambda i: (0, 0)),
            pl.BlockSpec((din, dout), lambda i: (0, 0)),
            pl.BlockSpec((1, dout), lambda i: (0, 0)),
        ],
        out_specs=[
            pl.BlockSpec((_R, dout), lambda i: (i, 0)),
            pl.BlockSpec((8, dout), lambda i: (0, 0)),
        ],
        out_shape=[
            jax.ShapeDtypeStruct((N, dout), jnp.float32),
            jax.ShapeDtypeStruct((8, dout), jnp.float32),
        ],
    )(agg, xin, wr, wo, b)


@functools.partial(jax.jit, static_argnames=("ow",))
def _bn_relu(y, st, g, be, ow):
    dout = y.shape[1]

    def body(y_ref, st_ref, g_ref, be_ref, h_ref):
        m = st_ref[0:1, :] * (1.0 / N)
        ex2 = st_ref[1:2, :] * (1.0 / N)
        inv = lax.rsqrt(jnp.maximum(ex2 - m * m, 0.0) + 1e-5)
        h = jnp.maximum(
            (y_ref[...] - m) * (inv * g_ref[...]) + be_ref[...], 0.0)
        if ow > dout:
            h = jnp.concatenate(
                [h, jnp.zeros((_R, ow - dout), jnp.float32)], axis=1)
        h_ref[...] = h

    return pl.pallas_call(
        body,
        grid=(N // _R,),
        in_specs=[
            pl.BlockSpec((_R, dout), lambda i: (i, 0)),
            pl.BlockSpec((8, dout), lambda i: (0, 0)),
            pl.BlockSpec((1, dout), lambda i: (0, 0)),
            pl.BlockSpec((1, dout), lambda i: (0, 0)),
        ],
        out_specs=pl.BlockSpec((_R, ow), lambda i: (i, 0)),
        out_shape=jax.ShapeDtypeStruct((N, ow), jnp.float32),
    )(y, st, g, be)


@jax.jit
def _bn_relu_head(y, st, g, be, wh1, bh1, wh2, bh2):
    dout = y.shape[1]

    def body(y_ref, st_ref, g_ref, be_ref, w1_ref, b1_ref, w2_ref, b2_ref,
             o_ref):
        m = st_ref[0:1, :] * (1.0 / N)
        ex2 = st_ref[1:2, :] * (1.0 / N)
        inv = lax.rsqrt(jnp.maximum(ex2 - m * m, 0.0) + 1e-5)
        h = jnp.maximum(
            (y_ref[...] - m) * (inv * g_ref[...]) + be_ref[...], 0.0)
        t = jnp.maximum(
            jnp.dot(h, w1_ref[...], preferred_element_type=jnp.float32)
            + b1_ref[...], 0.0)
        o_ref[...] = (jnp.dot(t, w2_ref[...],
                              preferred_element_type=jnp.float32)
                      + b2_ref[...])

    return pl.pallas_call(
        body,
        grid=(N // _R,),
        in_specs=[
            pl.BlockSpec((_R, dout), lambda i: (i, 0)),
            pl.BlockSpec((8, dout), lambda i: (0, 0)),
            pl.BlockSpec((1, dout), lambda i: (0, 0)),
            pl.BlockSpec((1, dout), lambda i: (0, 0)),
            pl.BlockSpec((dout, 128), lambda i: (0, 0)),
            pl.BlockSpec((1, 128), lambda i: (0, 0)),
            pl.BlockSpec((128, 1), lambda i: (0, 0)),
            pl.BlockSpec((1, 1), lambda i: (0, 0)),
        ],
        out_specs=pl.BlockSpec((_R, 1), lambda i: (i, 0)),
        out_shape=jax.ShapeDtypeStruct((N, 1), jnp.float32),
    )(y, st, g, be, wh1, bh1, wh2, bh2)


def kernel(x, edge_index, W_rel1, b_rel1, W_root1, g1, be1, W_rel2, b_rel2,
           W_root2, g2, be2, W_rel3, b_rel3, W_root3, g3, be3, Wh1, bh1,
           Wh2, bh2):
    src = edge_index[0].astype(jnp.int32)
    dst = edge_index[1].astype(jnp.int32)

    x128 = jnp.pad(x, ((0, 0), (0, 122)))
    wr1 = jnp.pad(W_rel1, ((0, 122), (0, 0)))
    wo1 = jnp.pad(W_root1, ((0, 122), (0, 0)))
    wr2 = jnp.pad(W_rel2, ((0, 64), (0, 0)))
    wo2 = jnp.pad(W_root2, ((0, 64), (0, 0)))

    agg1 = _segsum(x128, src, dst)
    y1, st1 = _mm_stats(agg1, x128, wr1, wo1, b_rel1.reshape(1, -1))
    h1 = _bn_relu(y1, st1, g1.reshape(1, -1), be1.reshape(1, -1), ow=128)

    agg2 = _segsum(h1, src, dst)
    y2, st2 = _mm_stats(agg2, h1, wr2, wo2, b_rel2.reshape(1, -1))
    h2 = _bn_relu(y2, st2, g2.reshape(1, -1), be2.reshape(1, -1), ow=128)

    agg3 = _segsum(h2, src, dst)
    y3, st3 = _mm_stats(agg3, h2, W_rel3, W_root3, b_rel3.reshape(1, -1))
    out = _bn_relu_head(y3, st3, g3.reshape(1, -1), be3.reshape(1, -1),
                        Wh1, bh1.reshape(1, -1), Wh2, bh2.reshape(1, -1))
    return out[:, 0]


# speedup vs baseline: 5.3129x; 1.2875x over previous
import functools

import jax
import jax.numpy as jnp
from jax import lax
from jax.experimental import pallas as pl
from jax.experimental.pallas import tpu as pltpu
from jax.experimental.pallas import tpu_sc as plsc

N = 50000
E = 800000
NSUB = 16
G = 128
EB = 2000


def _ceil(a, b):
    return -(-a // b)


@functools.partial(jax.jit, static_argnames=("d", "C", "stage"))
def _segsum(table, src, dst, d=128, C=6, stage=False):
    EBL = 1000 if stage else EB
    NB = 128 * _ceil(N, 128 * C)
    NPAD = NB * C
    CPS = C // 2
    EW = E // NSUB
    NBLK = EW // EBL
    VPB = EBL // 16
    ACC_ROWS = NB if stage else NB + 128
    ZRP = ACC_ROWS // NSUB
    RP = NB // NSUB
    TROWS = table.shape[0]
    TRP = TROWS // NSUB

    mesh = plsc.VectorSubcoreMesh(core_axis_name="c", subcore_axis_name="s")

    @functools.partial(
        pl.kernel,
        out_type=jax.ShapeDtypeStruct((NPAD, d), jnp.float32),
        mesh=mesh,
        compiler_params=pltpu.CompilerParams(needs_layout_passes=False),
        scratch_types=[
            pltpu.VMEM_SHARED((ACC_ROWS, d), jnp.float32),
            (pltpu.VMEM_SHARED((TROWS, d), jnp.float32)
             if stage else pltpu.VMEM((8,), jnp.float32)),
            pltpu.VMEM((EBL,), jnp.int32),
            pltpu.VMEM((EBL,), jnp.int32),
            pltpu.VMEM((EBL + G,), jnp.int32),
            pltpu.VMEM((EBL + G,), jnp.int32),
            pltpu.VMEM((1, G), jnp.int32),
            pltpu.VMEM((1, G), jnp.int32),
            pltpu.VMEM((1, G), jnp.int32),
            pltpu.VMEM((G, d), jnp.float32),
            pltpu.VMEM((G, d), jnp.float32),
            pltpu.VMEM((G, d), jnp.float32),
            pltpu.VMEM((16, d), jnp.float32),
            pltpu.SemaphoreType.DMA,
        ],
    )
    def seg(table_h, src_h, dst_h, out_h, acc, table_s, dst_buf, src_buf,
            src_list, loc_list, grp_idx0, grp_idx1, grp_idx2,
            rows0, rows1, rows2, zeros, gsem):
        rows_b = (rows0, rows1, rows2)
        grps_b = (grp_idx0, grp_idx1, grp_idx2)
        cid = lax.axis_index("c")
        sid = lax.axis_index("s")
        iota = lax.iota(jnp.int32, 16)
        z16 = jnp.zeros((16,), jnp.float32)

        if not stage:
            def zero_row(r, _):
                for j in range(d // 16):
                    zeros[r, pl.ds(j * 16, 16)] = z16
                return 0

            lax.fori_loop(0, 16, zero_row, 0)
            pad_src = cid * 256 + sid * 16 + iota
            pad_loc = NB + iota
        else:
            pltpu.sync_copy(table_h.at[pl.ds(sid * TRP, TRP)],
                            table_s.at[pl.ds(sid * TRP, TRP)])
            pad_src = N + iota
            pad_loc = iota
        gsrc = table_s if stage else table_h

        for k in range(CPS):
            lo = (cid * CPS + k) * NB
            hi = lo + NB

            zbase = sid * ZRP
            if stage:
                ZC = TROWS - N
                def zc(t, _):
                    pltpu.sync_copy(table_h.at[pl.ds(N, ZC)],
                                    acc.at[pl.ds(zbase + t * ZC, ZC)])
                    return 0
                lax.fori_loop(0, ZRP // ZC, zc, 0)
                if ZRP % ZC:
                    pltpu.sync_copy(
                        table_h.at[pl.ds(N, ZRP % ZC)],
                        acc.at[pl.ds(zbase + (ZRP // ZC) * ZC, ZRP % ZC)])
            else:
                def zc16(t, _):
                    pltpu.sync_copy(zeros,
                                    acc.at[pl.ds(zbase + t * 16, 16)])
                    return 0
                lax.fori_loop(0, ZRP // 16, zc16, 0)
                if ZRP % 16:
                    pltpu.sync_copy(
                        zeros.at[pl.ds(0, ZRP % 16)],
                        acc.at[pl.ds(zbase + (ZRP // 16) * 16, ZRP % 16)])
            plsc.subcore_barrier()

            def blk_body(blk, _):
                pob = 0
                base = sid * EW + blk * EBL
                pltpu.sync_copy(dst_h.at[pl.ds(base, EBL)], dst_buf)
                pltpu.sync_copy(src_h.at[pl.ds(base, EBL)], src_buf)

                def scan_body(i, cnt):
                    dv = dst_buf[pl.ds(pob + i * 16, 16)]
                    sv = src_buf[pl.ds(pob + i * 16, 16)]
                    m = (dv >= lo) & (dv < hi)
                    mi = m.astype(jnp.int32)
                    pos = cnt + plsc.cumsum(mi) - 1
                    plsc.store_scatter(src_list, [pos], sv, mask=m)
                    plsc.store_scatter(loc_list, [pos], dv - lo, mask=m)
                    return cnt + jnp.sum(mi)

                kcnt = lax.fori_loop(0, VPB, scan_body, 0)

                for t in range(G // 16):
                    src_list[pl.ds(kcnt + t * 16, 16)] = pad_src
                    loc_list[pl.ds(kcnt + t * 16, 16)] = pad_loc

                ngrp = (kcnt + G - 1) // G

                def quad_body(q, _):
                    for b in range(3):
                        g = q * 3 + b

                        @pl.when(g < ngrp)
                        def _(b=b, g=g):
                            pltpu.async_copy(
                                gsrc.at[src_list.at[pl.ds(g * G, G)]],
                                rows_b[b], gsem)
                    for b in range(3):
                        g = q * 3 + b

                        @pl.when(g < ngrp)
                        def _(b=b, g=g):
                            pltpu.make_async_copy(
                                gsrc.at[src_list.at[pl.ds(g * G, G)]],
                                rows_b[b], gsem).wait()
                            for t in range(G // 16):
                                grps_b[b][0, pl.ds(t * 16, 16)] = (
                                    loc_list[pl.ds(g * G + t * 16, 16)])
                    for b in range(3):
                        g = q * 3 + b

                        @pl.when(g < ngrp)
                        def _(b=b, g=g):
                            pltpu.sync_copy(rows_b[b],
                                            acc.at[grps_b[b].at[0]],
                                            add=True)
                    return 0

                lax.fori_loop(0, (ngrp + 2) // 3, quad_body, 0)
                return 0

            lax.fori_loop(0, NBLK, blk_body, 0)
            plsc.subcore_barrier()

            pltpu.sync_copy(acc.at[pl.ds(sid * RP, RP)],
                            out_h.at[pl.ds(lo + sid * RP, RP)])
            plsc.subcore_barrier()

    return seg(table, src, dst)


_R = 2000


@jax.jit
def _mm_stats(agg, xin, wr, wo, b):
    dout = wr.shape[1]

    def body(agg_ref, x_ref, wr_ref, wo_ref, b_ref, y_ref, st_ref):
        yb = jnp.dot(agg_ref[...], wr_ref[...],
                     preferred_element_type=jnp.float32)
        yb = yb + jnp.dot(x_ref[...], wo_ref[...],
                          preferred_element_type=jnp.float32)
        yb = yb + b_ref[...]
        y_ref[...] = yb
        s = jnp.sum(yb, axis=0, keepdims=True)
        s2 = jnp.sum(yb * yb, axis=0, keepdims=True)
        st = jnp.concatenate(
            [s, s2, jnp.zeros((6, dout), jnp.float32)], axis=0)

        @pl.when(pl.program_id(0) == 0)
        def _():
            st_ref[...] = st

        @pl.when(pl.program_id(0) > 0)
        def _():
            st_ref[...] = st_ref[...] + st

    din = agg.shape[1]
    return pl.pallas_call(
        body,
        grid=(N // _R,),
        in_specs=[
            pl.BlockSpec((_R, din), lambda i: (i, 0)),
            pl.BlockSpec((_R, din), lambda i: (i, 0)),
            pl.BlockSpec((din, dout), lambda i: (0, 0)),
            pl.BlockSpec((din, dout), lambda i: (0, 0)),
            pl.BlockSpec((1, dout), lambda i: (0, 0)),
        ],
        out_specs=[
            pl.BlockSpec((_R, dout), lambda i: (i, 0)),
            pl.BlockSpec((8, dout), lambda i: (0, 0)),
        ],
        out_shape=[
            jax.ShapeDtypeStruct((N, dout), jnp.float32),
            jax.ShapeDtypeStruct((8, dout), jnp.float32),
        ],
    )(agg, xin, wr, wo, b)


@functools.partial(jax.jit, static_argnames=("ow",))
def _bn_relu(y, st, g, be, ow):
    dout = y.shape[1]

    def body(y_ref, st_ref, g_ref, be_ref, h_ref):
        m = st_ref[0:1, :] * (1.0 / N)
        ex2 = st_ref[1:2, :] * (1.0 / N)
        inv = lax.rsqrt(jnp.maximum(ex2 - m * m, 0.0) + 1e-5)
        h = jnp.maximum(
            (y_ref[...] - m) * (inv * g_ref[...]) + be_ref[...], 0.0)
        if ow > dout:
            h = jnp.concatenate(
                [h, jnp.zeros((_R, ow - dout), jnp.float32)], axis=1)
        h_ref[...] = h

    return pl.pallas_call(
        body,
        grid=(N // _R,),
        in_specs=[
            pl.BlockSpec((_R, dout), lambda i: (i, 0)),
            pl.BlockSpec((8, dout), lambda i: (0, 0)),
            pl.BlockSpec((1, dout), lambda i: (0, 0)),
            pl.BlockSpec((1, dout), lambda i: (0, 0)),
        ],
        out_specs=pl.BlockSpec((_R, ow), lambda i: (i, 0)),
        out_shape=jax.ShapeDtypeStruct((N, ow), jnp.float32),
    )(y, st, g, be)


@jax.jit
def _bn_relu_head(y, st, g, be, wh1, bh1, wh2, bh2):
    dout = y.shape[1]

    def body(y_ref, st_ref, g_ref, be_ref, w1_ref, b1_ref, w2_ref, b2_ref,
             o_ref):
        m = st_ref[0:1, :] * (1.0 / N)
        ex2 = st_ref[1:2, :] * (1.0 / N)
        inv = lax.rsqrt(jnp.maximum(ex2 - m * m, 0.0) + 1e-5)
        h = jnp.maximum(
            (y_ref[...] - m) * (inv * g_ref[...]) + be_ref[...], 0.0)
        t = jnp.maximum(
            jnp.dot(h, w1_ref[...], preferred_element_type=jnp.float32)
            + b1_ref[...], 0.0)
        o_ref[...] = (jnp.dot(t, w2_ref[...],
                              preferred_element_type=jnp.float32)
                      + b2_ref[...])

    return pl.pallas_call(
        body,
        grid=(N // _R,),
        in_specs=[
            pl.BlockSpec((_R, dout), lambda i: (i, 0)),
            pl.BlockSpec((8, dout), lambda i: (0, 0)),
            pl.BlockSpec((1, dout), lambda i: (0, 0)),
            pl.BlockSpec((1, dout), lambda i: (0, 0)),
            pl.BlockSpec((dout, 128), lambda i: (0, 0)),
            pl.BlockSpec((1, 128), lambda i: (0, 0)),
            pl.BlockSpec((128, 1), lambda i: (0, 0)),
            pl.BlockSpec((1, 1), lambda i: (0, 0)),
        ],
        out_specs=pl.BlockSpec((_R, 1), lambda i: (i, 0)),
        out_shape=jax.ShapeDtypeStruct((N, 1), jnp.float32),
    )(y, st, g, be, wh1, bh1, wh2, bh2)


def kernel(x, edge_index, W_rel1, b_rel1, W_root1, g1, be1, W_rel2, b_rel2,
           W_root2, g2, be2, W_rel3, b_rel3, W_root3, g3, be3, Wh1, bh1,
           Wh2, bh2):
    src = edge_index[0].astype(jnp.int32)
    dst = edge_index[1].astype(jnp.int32)

    x128 = jnp.pad(x, ((0, 0), (0, 122)))
    wr1 = jnp.pad(W_rel1, ((0, 122), (0, 0)))
    wo1 = jnp.pad(W_root1, ((0, 122), (0, 0)))
    wr2 = jnp.pad(W_rel2, ((0, 64), (0, 0)))
    wo2 = jnp.pad(W_root2, ((0, 64), (0, 0)))

    agg1 = _segsum(x128, src, dst)
    y1, st1 = _mm_stats(agg1, x128, wr1, wo1, b_rel1.reshape(1, -1))
    h1 = _bn_relu(y1, st1, g1.reshape(1, -1), be1.reshape(1, -1), ow=128)

    agg2 = _segsum(h1, src, dst)
    y2, st2 = _mm_stats(agg2, h1, wr2, wo2, b_rel2.reshape(1, -1))
    h2 = _bn_relu(y2, st2, g2.reshape(1, -1), be2.reshape(1, -1), ow=128)

    agg3 = _segsum(h2, src, dst)
    y3, st3 = _mm_stats(agg3, h2, W_rel3, W_root3, b_rel3.reshape(1, -1))
    out = _bn_relu_head(y3, st3, g3.reshape(1, -1), be3.reshape(1, -1),
                        Wh1, bh1.reshape(1, -1), Wh2, bh2.reshape(1, -1))
    return out[:, 0]


# speedup vs baseline: 6.9716x; 1.3122x over previous
import functools

import jax
import jax.numpy as jnp
from jax import lax
from jax.experimental import pallas as pl
from jax.experimental.pallas import tpu as pltpu
from jax.experimental.pallas import tpu_sc as plsc

N = 50000
E = 800000
NSUB = 16
G = 128
EB = 2000


def _ceil(a, b):
    return -(-a // b)


@functools.partial(jax.jit, static_argnames=("d", "C", "stage"))
def _segsum(table, src, dst, zrows, d=128, C=6, stage=False):
    EBL = 1000 if stage else EB
    NB = 128 * _ceil(N, 128 * C)
    NPAD = NB * C
    CPS = C // 2
    EW = E // NSUB
    NBLK = EW // EBL
    VPB = EBL // 16
    ACC_ROWS = NB if stage else NB + 128
    ZRP = ACC_ROWS // NSUB
    RP = NB // NSUB
    TROWS = table.shape[0]
    TRP = TROWS // NSUB

    mesh = plsc.VectorSubcoreMesh(core_axis_name="c", subcore_axis_name="s")

    @functools.partial(
        pl.kernel,
        out_type=jax.ShapeDtypeStruct((NPAD, d), jnp.float32),
        mesh=mesh,
        compiler_params=pltpu.CompilerParams(needs_layout_passes=False),
        scratch_types=[
            pltpu.VMEM_SHARED((ACC_ROWS, d), jnp.float32),
            (pltpu.VMEM_SHARED((TROWS, d), jnp.float32)
             if stage else pltpu.VMEM((8,), jnp.float32)),
            pltpu.VMEM((2 * EBL,), jnp.int32),
            pltpu.VMEM((2 * EBL,), jnp.int32),
            pltpu.VMEM((EBL + G,), jnp.int32),
            pltpu.VMEM((EBL + G,), jnp.int32),
            pltpu.VMEM((1, G), jnp.int32),
            pltpu.VMEM((1, G), jnp.int32),
            pltpu.VMEM((1, G), jnp.int32),
            pltpu.VMEM((G, d), jnp.float32),
            pltpu.VMEM((G, d), jnp.float32),
            pltpu.VMEM((G, d), jnp.float32),
            pltpu.SMEM((128,), jnp.int32),
            pltpu.SemaphoreType.DMA,
            pltpu.SemaphoreType.DMA,
            pltpu.SemaphoreType.DMA,
            pltpu.SemaphoreType.DMA,
            pltpu.SemaphoreType.DMA,
        ],
    )
    def seg(table_h, src_h, dst_h, zr_h, out_h, acc, table_s, dst_buf,
            src_buf, src_list, loc_list, grp_idx0, grp_idx1, grp_idx2,
            rows0, rows1, rows2, cnts, gsem, isem, ssem0, ssem1, ssem2):
        rows_b = (rows0, rows1, rows2)
        grps_b = (grp_idx0, grp_idx1, grp_idx2)
        ssems = (ssem0, ssem1, ssem2)
        cid = lax.axis_index("c")
        sid = lax.axis_index("s")
        iota = lax.iota(jnp.int32, 16)

        if not stage:
            pad_src = cid * 256 + sid * 16 + iota
            pad_loc = NB + iota
        else:
            pltpu.sync_copy(table_h.at[pl.ds(sid * TRP, TRP)],
                            table_s.at[pl.ds(sid * TRP, TRP)])
            pad_src = N + iota
            pad_loc = iota
        gsrc = table_s if stage else table_h

        for k in range(CPS):
            lo = (cid * CPS + k) * NB
            hi = lo + NB

            zbase = sid * ZRP
            if stage:
                ZC = TROWS - N
                def zc(t, _):
                    pltpu.sync_copy(table_h.at[pl.ds(N, ZC)],
                                    acc.at[pl.ds(zbase + t * ZC, ZC)])
                    return 0
                lax.fori_loop(0, ZRP // ZC, zc, 0)
                if ZRP % ZC:
                    pltpu.sync_copy(
                        table_h.at[pl.ds(N, ZRP % ZC)],
                        acc.at[pl.ds(zbase + (ZRP // ZC) * ZC, ZRP % ZC)])
            else:
                def zc128(t, _):
                    pltpu.sync_copy(zr_h,
                                    acc.at[pl.ds(zbase + t * 128, 128)])
                    return 0
                lax.fori_loop(0, ZRP // 128, zc128, 0)
                if ZRP % 128:
                    pltpu.sync_copy(
                        zr_h.at[pl.ds(0, ZRP % 128)],
                        acc.at[pl.ds(zbase + (ZRP // 128) * 128, ZRP % 128)])
            plsc.subcore_barrier()

            def drain_scat(prev_ngrp):
                for b in range(3):
                    @pl.when(prev_ngrp > b)
                    def _(b=b):
                        pltpu.make_async_copy(
                            rows_b[b], acc.at[grps_b[b].at[0]],
                            ssems[b]).wait()

            def load_idx(blk, par):
                base = sid * EW + blk * EBL
                pltpu.async_copy(dst_h.at[pl.ds(base, EBL)],
                                 dst_buf.at[pl.ds(par * EBL, EBL)], isem)
                pltpu.async_copy(src_h.at[pl.ds(base, EBL)],
                                 src_buf.at[pl.ds(par * EBL, EBL)], isem)

            def wait_idx(blk, par):
                base = sid * EW + blk * EBL
                pltpu.make_async_copy(
                    dst_h.at[pl.ds(base, EBL)],
                    dst_buf.at[pl.ds(par * EBL, EBL)], isem).wait()
                pltpu.make_async_copy(
                    src_h.at[pl.ds(base, EBL)],
                    src_buf.at[pl.ds(par * EBL, EBL)], isem).wait()

            load_idx(0, 0)

            def blk_body(blk, prev_ngrp):
                par = blk % 2
                pob = par * EBL
                wait_idx(blk, par)

                @pl.when(blk + 1 < NBLK)
                def _():
                    load_idx(blk + 1, 1 - par)

                @plsc.parallel_loop(0, VPB, unroll=4)
                def _(i):
                    dv = dst_buf[pl.ds(pob + i * 16, 16)]
                    m = (dv >= lo) & (dv < hi)
                    cnts[i] = jnp.sum(m.astype(jnp.int32))

                def pfx_body(i, run):
                    c = cnts[i]
                    cnts[i] = run
                    return run + c

                kcnt = lax.fori_loop(0, VPB, pfx_body, 0)

                @plsc.parallel_loop(0, VPB, unroll=4)
                def _(i):
                    dv = dst_buf[pl.ds(pob + i * 16, 16)]
                    sv = src_buf[pl.ds(pob + i * 16, 16)]
                    m = (dv >= lo) & (dv < hi)
                    pos = cnts[i] + plsc.cumsum(m.astype(jnp.int32)) - 1
                    plsc.store_scatter(src_list, [pos], sv, mask=m)
                    plsc.store_scatter(loc_list, [pos], dv - lo, mask=m)

                for t in range(G // 16):
                    src_list[pl.ds(kcnt + t * 16, 16)] = pad_src
                    loc_list[pl.ds(kcnt + t * 16, 16)] = pad_loc

                ngrp = (kcnt + G - 1) // G
                drain_scat(prev_ngrp)

                def quad_body(q, _):
                    for b in range(3):
                        g = q * 3 + b

                        @pl.when((g < ngrp) & (g >= 3))
                        def _(b=b, g=g):
                            pltpu.make_async_copy(
                                rows_b[b], acc.at[grps_b[b].at[0]],
                                ssems[b]).wait()

                        @pl.when(g < ngrp)
                        def _(b=b, g=g):
                            pltpu.async_copy(
                                gsrc.at[src_list.at[pl.ds(g * G, G)]],
                                rows_b[b], gsem)
                    for b in range(3):
                        g = q * 3 + b

                        @pl.when(g < ngrp)
                        def _(b=b, g=g):
                            pltpu.make_async_copy(
                                gsrc.at[src_list.at[pl.ds(g * G, G)]],
                                rows_b[b], gsem).wait()
                            for t in range(G // 16):
                                grps_b[b][0, pl.ds(t * 16, 16)] = (
                                    loc_list[pl.ds(g * G + t * 16, 16)])
                    for b in range(3):
                        g = q * 3 + b

                        @pl.when(g < ngrp)
                        def _(b=b, g=g):
                            pltpu.async_copy(rows_b[b],
                                             acc.at[grps_b[b].at[0]],
                                             ssems[b], add=True)
                    return 0

                lax.fori_loop(0, (ngrp + 2) // 3, quad_body, 0)
                return ngrp

            last_ngrp = lax.fori_loop(0, NBLK, blk_body, 0)
            drain_scat(last_ngrp)
            plsc.subcore_barrier()

            pltpu.sync_copy(acc.at[pl.ds(sid * RP, RP)],
                            out_h.at[pl.ds(lo + sid * RP, RP)])
            plsc.subcore_barrier()

    return seg(table, src, dst, zrows)


_R = 2000


@jax.jit
def _root_mm(h, wo, b):
    din, dout = wo.shape

    def body(h_ref, wo_ref, b_ref, r_ref):
        r_ref[...] = jnp.dot(h_ref[...], wo_ref[...],
                             preferred_element_type=jnp.float32) + b_ref[...]

    return pl.pallas_call(
        body,
        grid=(N // _R,),
        in_specs=[
            pl.BlockSpec((_R, din), lambda i: (i, 0)),
            pl.BlockSpec((din, dout), lambda i: (0, 0)),
            pl.BlockSpec((1, dout), lambda i: (0, 0)),
        ],
        out_specs=pl.BlockSpec((_R, dout), lambda i: (i, 0)),
        out_shape=jax.ShapeDtypeStruct((N, dout), jnp.float32),
    )(h, wo, b)


@jax.jit
def _rel_stats(agg, r, wr):
    din, dout = wr.shape

    def body(agg_ref, r_ref, wr_ref, y_ref, st_ref):
        yb = jnp.dot(agg_ref[...], wr_ref[...],
                     preferred_element_type=jnp.float32) + r_ref[...]
        y_ref[...] = yb
        s = jnp.sum(yb, axis=0, keepdims=True)
        s2 = jnp.sum(yb * yb, axis=0, keepdims=True)
        st = jnp.concatenate(
            [s, s2, jnp.zeros((6, dout), jnp.float32)], axis=0)

        @pl.when(pl.program_id(0) == 0)
        def _():
            st_ref[...] = st

        @pl.when(pl.program_id(0) > 0)
        def _():
            st_ref[...] = st_ref[...] + st

    return pl.pallas_call(
        body,
        grid=(N // _R,),
        in_specs=[
            pl.BlockSpec((_R, din), lambda i: (i, 0)),
            pl.BlockSpec((_R, dout), lambda i: (i, 0)),
            pl.BlockSpec((din, dout), lambda i: (0, 0)),
        ],
        out_specs=[
            pl.BlockSpec((_R, dout), lambda i: (i, 0)),
            pl.BlockSpec((8, dout), lambda i: (0, 0)),
        ],
        out_shape=[
            jax.ShapeDtypeStruct((N, dout), jnp.float32),
            jax.ShapeDtypeStruct((8, dout), jnp.float32),
        ],
    )(agg, r, wr)


@functools.partial(jax.jit, static_argnames=("ow",))
def _bn_relu(y, st, g, be, ow):
    dout = y.shape[1]

    def body(y_ref, st_ref, g_ref, be_ref, h_ref):
        m = st_ref[0:1, :] * (1.0 / N)
        ex2 = st_ref[1:2, :] * (1.0 / N)
        inv = lax.rsqrt(jnp.maximum(ex2 - m * m, 0.0) + 1e-5)
        h = jnp.maximum(
            (y_ref[...] - m) * (inv * g_ref[...]) + be_ref[...], 0.0)
        if ow > dout:
            h = jnp.concatenate(
                [h, jnp.zeros((_R, ow - dout), jnp.float32)], axis=1)
        h_ref[...] = h

    return pl.pallas_call(
        body,
        grid=(N // _R,),
        in_specs=[
            pl.BlockSpec((_R, dout), lambda i: (i, 0)),
            pl.BlockSpec((8, dout), lambda i: (0, 0)),
            pl.BlockSpec((1, dout), lambda i: (0, 0)),
            pl.BlockSpec((1, dout), lambda i: (0, 0)),
        ],
        out_specs=pl.BlockSpec((_R, ow), lambda i: (i, 0)),
        out_shape=jax.ShapeDtypeStruct((N, ow), jnp.float32),
    )(y, st, g, be)


@jax.jit
def _bn_relu_head(y, st, g, be, wh1, bh1, wh2, bh2):
    dout = y.shape[1]

    def body(y_ref, st_ref, g_ref, be_ref, w1_ref, b1_ref, w2_ref, b2_ref,
             o_ref):
        m = st_ref[0:1, :] * (1.0 / N)
        ex2 = st_ref[1:2, :] * (1.0 / N)
        inv = lax.rsqrt(jnp.maximum(ex2 - m * m, 0.0) + 1e-5)
        h = jnp.maximum(
            (y_ref[...] - m) * (inv * g_ref[...]) + be_ref[...], 0.0)
        t = jnp.maximum(
            jnp.dot(h, w1_ref[...], preferred_element_type=jnp.float32)
            + b1_ref[...], 0.0)
        o_ref[...] = (jnp.dot(t, w2_ref[...],
                              preferred_element_type=jnp.float32)
                      + b2_ref[...])

    return pl.pallas_call(
        body,
        grid=(N // _R,),
        in_specs=[
            pl.BlockSpec((_R, dout), lambda i: (i, 0)),
            pl.BlockSpec((8, dout), lambda i: (0, 0)),
            pl.BlockSpec((1, dout), lambda i: (0, 0)),
            pl.BlockSpec((1, dout), lambda i: (0, 0)),
            pl.BlockSpec((dout, 128), lambda i: (0, 0)),
            pl.BlockSpec((1, 128), lambda i: (0, 0)),
            pl.BlockSpec((128, 1), lambda i: (0, 0)),
            pl.BlockSpec((1, 1), lambda i: (0, 0)),
        ],
        out_specs=pl.BlockSpec((_R, 1), lambda i: (i, 0)),
        out_shape=jax.ShapeDtypeStruct((N, 1), jnp.float32),
    )(y, st, g, be, wh1, bh1, wh2, bh2)


def kernel(x, edge_index, W_rel1, b_rel1, W_root1, g1, be1, W_rel2, b_rel2,
           W_root2, g2, be2, W_rel3, b_rel3, W_root3, g3, be3, Wh1, bh1,
           Wh2, bh2):
    src = edge_index[0].astype(jnp.int32)
    dst = edge_index[1].astype(jnp.int32)

    x16 = jnp.pad(x, ((0, 0), (0, 10)))
    x128 = jnp.pad(x16, ((0, 0), (0, 112)))
    wr1 = jnp.pad(W_rel1, ((0, 122), (0, 0)))
    wo1 = jnp.pad(W_root1, ((0, 10), (0, 0)))
    wr2 = jnp.pad(W_rel2, ((0, 64), (0, 0)))
    wo2 = jnp.pad(W_root2, ((0, 64), (0, 0)))

    zrows = jnp.zeros((128, 128), jnp.float32)

    r1 = _root_mm(x16, wo1, b_rel1.reshape(1, -1))
    agg1 = _segsum(x128, src, dst, zrows)
    y1, st1 = _rel_stats(agg1, r1, wr1)
    h1 = _bn_relu(y1, st1, g1.reshape(1, -1), be1.reshape(1, -1), ow=128)

    r2 = _root_mm(h1, wo2, b_rel2.reshape(1, -1))
    agg2 = _segsum(h1, src, dst, zrows)
    y2, st2 = _rel_stats(agg2, r2, wr2)
    h2 = _bn_relu(y2, st2, g2.reshape(1, -1), be2.reshape(1, -1), ow=128)

    r3 = _root_mm(h2, W_root3, b_rel3.reshape(1, -1))
    agg3 = _segsum(h2, src, dst, zrows)
    y3, st3 = _rel_stats(agg3, r3, W_rel3)
    out = _bn_relu_head(y3, st3, g3.reshape(1, -1), be3.reshape(1, -1),
                        Wh1, bh1.reshape(1, -1), Wh2, bh2.reshape(1, -1))
    return out[:, 0]
